# baseline (speedup 1.0000x reference)
import functools

import numpy as np
import jax
import jax.numpy as jnp
from jax import lax
from jax.experimental import pallas as pl
from jax.experimental.pallas import tpu as pltpu
from jax.experimental.pallas import tpu_sc as plsc

N = 2048
E = 65536
D = 128
L = 5

NUM_CORES = 2
NUM_SUBCORES = 16
NW = NUM_CORES * NUM_SUBCORES
TR = N // 8
TR_PER_W = TR // NW
LC = 512
LCHUNKS = N // LC
CHUNKS = TR_PER_W * LCHUNKS
VECS = 8 * LC // 16
VPR_SHIFT = 5
NBUF = 4
OBUF = 2

_HI_MASK = np.int32(-65536)


def _scores_body(ev_ref, ea_ref, out_ref):
    out_ref[...] = lax.dot_general(
        ev_ref[...], ea_ref[...],
        (((1,), (1,)), ((), ())),
        preferred_element_type=jnp.float32,
    )


def _tc_scores(ev_pad, edge_attr):
    blk = 8192
    return pl.pallas_call(
        _scores_body,
        grid=(E // blk,),
        in_specs=[
            pl.BlockSpec((8, D), lambda j: (0, 0)),
            pl.BlockSpec((blk, D), lambda j: (j, 0)),
        ],
        out_specs=pl.BlockSpec((8, blk), lambda j: (0, j)),
        out_shape=jax.ShapeDtypeStruct((8, E), jnp.float32),
    )(ev_pad, edge_attr)


def _unpack_lo(w):
    return plsc.bitcast(lax.shift_left(w, 16), jnp.float32)


def _unpack_hi(w):
    return plsc.bitcast(lax.bitwise_and(w, _HI_MASK), jnp.float32)


def _sc_body(ep_ref, t01_ref, t23_ref, t4_ref, out_ref,
             tab_ref, inb_ref, prev_ref, outb_ref,
             sem_in0, sem_in1, sem_in2, sem_in3, sem_out0, sem_out1):
    wid = lax.axis_index("s") * NUM_CORES + lax.axis_index("c")
    row0 = wid * TR_PER_W * 8
    sems_in = (sem_in0, sem_in1, sem_in2, sem_in3)
    sems_out = (sem_out0, sem_out1)

    def slab(c):
        r8 = row0 + (c // LCHUNKS) * 8
        lo = (c % LCHUNKS) * LC
        return r8, lo

    def run_phase(planes, with_prev):
        def start_in(c, b):
            r8, lo = slab(c)
            for k, p in enumerate(planes):
                pltpu.async_copy(
                    ep_ref.at[p, pl.ds(r8, 8), pl.ds(lo, LC)],
                    inb_ref.at[b, k], sems_in[b])
            if with_prev:
                pltpu.async_copy(
                    out_ref.at[pl.ds(r8, 8), pl.ds(lo, LC)],
                    prev_ref.at[b], sems_in[b])

        def wait_in(c, b):
            r8, lo = slab(c)
            for k in range(len(planes)):
                pltpu.make_async_copy(
                    ep_ref.at[planes[0], pl.ds(r8, 8), pl.ds(lo, LC)],
                    inb_ref.at[b, k], sems_in[b]).wait()
            if with_prev:
                pltpu.make_async_copy(
                    out_ref.at[pl.ds(r8, 8), pl.ds(lo, LC)],
                    prev_ref.at[b], sems_in[b]).wait()

        def start_out(c, ob):
            r8, lo = slab(c)
            pltpu.async_copy(
                outb_ref.at[ob],
                out_ref.at[pl.ds(r8, 8), pl.ds(lo, LC)], sems_out[ob])

        def wait_out(c, ob):
            r8, lo = slab(c)
            pltpu.make_async_copy(
                outb_ref.at[ob],
                out_ref.at[pl.ds(r8, 8), pl.ds(lo, LC)], sems_out[ob]).wait()

        def compute(b, ob):
            @plsc.parallel_loop(0, VECS, unroll=16)
            def _(v):
                ri = lax.shift_right_logical(v, VPR_SHIFT)
                ci = lax.shift_left(
                    lax.bitwise_and(v, (1 << VPR_SHIFT) - 1), 4)
                if len(planes) == 2:
                    i0 = inb_ref[b, 0, ri, pl.ds(ci, 16)]
                    i1 = inb_ref[b, 1, ri, pl.ds(ci, 16)]
                    w0 = plsc.load_gather(tab_ref, [i0])
                    w1 = plsc.load_gather(tab_ref, [i1])
                    acc = _unpack_lo(w0) + _unpack_hi(w1)
                else:
                    i4 = inb_ref[b, 0, ri, pl.ds(ci, 16)]
                    acc = plsc.bitcast(
                        plsc.load_gather(tab_ref, [i4]), jnp.float32)
                if with_prev:
                    acc = acc + prev_ref[b, ri, pl.ds(ci, 16)]
                outb_ref[ob, ri, pl.ds(ci, 16)] = acc

        for b in range(NBUF):
            start_in(b, b)

        def main(cq, carry):
            for b in range(NBUF):
                c = NBUF * cq + b
                ob = b % OBUF
                wait_in(c, b)

                @pl.when(c >= OBUF)
                def _():
                    wait_out(c - OBUF, ob)

                compute(b, ob)
                start_out(c, ob)

                @pl.when(c + NBUF < CHUNKS)
                def _():
                    start_in(c + NBUF, b)
            return carry

        lax.fori_loop(0, CHUNKS // NBUF, main, 0)
        wait_out(CHUNKS - 2, (CHUNKS - 2) % OBUF)
        wait_out(CHUNKS - 1, (CHUNKS - 1) % OBUF)

    pltpu.sync_copy(t01_ref, tab_ref)
    run_phase((0, 1), False)

    pltpu.sync_copy(t23_ref, tab_ref)
    run_phase((2, 3), True)

    pltpu.sync_copy(t4_ref, tab_ref)
    run_phase((4,), True)


def _sc_gather(ep_t, t01, t23, t4):
    mesh = plsc.VectorSubcoreMesh(core_axis_name="c", subcore_axis_name="s")
    kern = functools.partial(
        pl.kernel,
        mesh=mesh,
        compiler_params=pltpu.CompilerParams(needs_layout_passes=False),
        out_type=jax.ShapeDtypeStruct((N, N), jnp.float32),
        scratch_types=[
            pltpu.VMEM((E,), jnp.int32),
            pltpu.VMEM((NBUF, 2, 8, LC), jnp.int32),
            pltpu.VMEM((NBUF, 8, LC), jnp.float32),
            pltpu.VMEM((OBUF, 8, LC), jnp.float32),
            pltpu.SemaphoreType.DMA,
            pltpu.SemaphoreType.DMA,
            pltpu.SemaphoreType.DMA,
            pltpu.SemaphoreType.DMA,
            pltpu.SemaphoreType.DMA,
            pltpu.SemaphoreType.DMA,
        ],
    )(_sc_body)
    return kern(ep_t, t01, t23, t4)


def kernel(x, edge_attr, edge_paths, edge_vector):
    assert edge_attr.shape == (E, D) and edge_paths.shape == (N, N, L)
    ev_pad = jnp.zeros((8, D), jnp.float32).at[:L].set(edge_vector / L)
    scores_t = _tc_scores(ev_pad, edge_attr)

    u = lax.bitcast_convert_type(
        scores_t.astype(jnp.bfloat16), jnp.uint16).astype(jnp.uint32)
    w01 = jnp.array([1, 1 << 16, 0, 0, 0, 0, 0, 0], jnp.uint32)
    w23 = jnp.array([0, 0, 1, 1 << 16, 0, 0, 0, 0], jnp.uint32)
    e4 = jnp.array([0, 0, 0, 0, 1, 0, 0, 0], jnp.float32)
    t01 = lax.bitcast_convert_type((u * w01[:, None]).sum(0), jnp.int32)
    t23 = lax.bitcast_convert_type((u * w23[:, None]).sum(0), jnp.int32)
    t4 = lax.bitcast_convert_type((scores_t * e4[:, None]).sum(0), jnp.int32)

    ep_t = jnp.transpose(edge_paths, (2, 0, 1))
    return _sc_gather(ep_t, t01, t23, t4)


# speedup vs baseline: 1.0014x; 1.0014x over previous
import functools

import numpy as np
import jax
import jax.numpy as jnp
from jax import lax
from jax.experimental import pallas as pl
from jax.experimental.pallas import tpu as pltpu
from jax.experimental.pallas import tpu_sc as plsc

N = 2048
E = 65536
D = 128
L = 5

NUM_CORES = 2
NUM_SUBCORES = 16
NW = NUM_CORES * NUM_SUBCORES
TR = N // 8
TR_PER_W = TR // NW
LC = 512
LCHUNKS = N // LC
CHUNKS = TR_PER_W * LCHUNKS
VECS = 8 * LC // 16
VPR_SHIFT = 5
NBUF = 4
OBUF = 2

_HI_MASK = np.int32(-65536)


def _scores_body(ev_ref, ea_ref, out_ref):
    out_ref[...] = lax.dot_general(
        ev_ref[...], ea_ref[...],
        (((1,), (1,)), ((), ())),
        preferred_element_type=jnp.float32,
    )


def _tc_scores(ev_pad, edge_attr):
    blk = 8192
    return pl.pallas_call(
        _scores_body,
        grid=(E // blk,),
        in_specs=[
            pl.BlockSpec((8, D), lambda j: (0, 0)),
            pl.BlockSpec((blk, D), lambda j: (j, 0)),
        ],
        out_specs=pl.BlockSpec((8, blk), lambda j: (0, j)),
        out_shape=jax.ShapeDtypeStruct((8, E), jnp.float32),
    )(ev_pad, edge_attr)


def _unpack_lo(w):
    return plsc.bitcast(lax.shift_left(w, 16), jnp.float32)


def _unpack_hi(w):
    return plsc.bitcast(lax.bitwise_and(w, _HI_MASK), jnp.float32)


def _sc_body(ep_ref, t01_ref, t23_ref, t4_ref, out_ref,
             tab_ref, inb_ref, prev_ref, outb_ref,
             sem_in0, sem_in1, sem_in2, sem_in3, sem_out0, sem_out1):
    wid = lax.axis_index("s") * NUM_CORES + lax.axis_index("c")
    row0 = wid * TR_PER_W * 8
    sems_in = (sem_in0, sem_in1, sem_in2, sem_in3)
    sems_out = (sem_out0, sem_out1)

    def slab(c):
        r8 = row0 + (c // LCHUNKS) * 8
        lo = (c % LCHUNKS) * LC
        return r8, lo

    def run_phase(planes, with_prev):
        def start_in(c, b):
            r8, lo = slab(c)
            for k, p in enumerate(planes):
                pltpu.async_copy(
                    ep_ref.at[p, pl.ds(r8, 8), pl.ds(lo, LC)],
                    inb_ref.at[b, k], sems_in[b])
            if with_prev:
                pltpu.async_copy(
                    out_ref.at[pl.ds(r8, 8), pl.ds(lo, LC)],
                    prev_ref.at[b], sems_in[b])

        def wait_in(c, b):
            r8, lo = slab(c)
            for k in range(len(planes)):
                pltpu.make_async_copy(
                    ep_ref.at[planes[0], pl.ds(r8, 8), pl.ds(lo, LC)],
                    inb_ref.at[b, k], sems_in[b]).wait()
            if with_prev:
                pltpu.make_async_copy(
                    out_ref.at[pl.ds(r8, 8), pl.ds(lo, LC)],
                    prev_ref.at[b], sems_in[b]).wait()

        def start_out(c, ob):
            r8, lo = slab(c)
            pltpu.async_copy(
                outb_ref.at[ob],
                out_ref.at[pl.ds(r8, 8), pl.ds(lo, LC)], sems_out[ob])

        def wait_out(c, ob):
            r8, lo = slab(c)
            pltpu.make_async_copy(
                outb_ref.at[ob],
                out_ref.at[pl.ds(r8, 8), pl.ds(lo, LC)], sems_out[ob]).wait()

        def compute(b, ob):
            @plsc.parallel_loop(0, VECS, unroll=8)
            def _(v):
                ri = lax.shift_right_logical(v, VPR_SHIFT)
                ci = lax.shift_left(
                    lax.bitwise_and(v, (1 << VPR_SHIFT) - 1), 4)
                if len(planes) == 2:
                    i0 = inb_ref[b, 0, ri, pl.ds(ci, 16)]
                    i1 = inb_ref[b, 1, ri, pl.ds(ci, 16)]
                    w0 = plsc.load_gather(tab_ref, [i0])
                    w1 = plsc.load_gather(tab_ref, [i1])
                    acc = _unpack_lo(w0) + _unpack_hi(w1)
                else:
                    i4 = inb_ref[b, 0, ri, pl.ds(ci, 16)]
                    acc = plsc.bitcast(
                        plsc.load_gather(tab_ref, [i4]), jnp.float32)
                if with_prev:
                    acc = acc + prev_ref[b, ri, pl.ds(ci, 16)]
                outb_ref[ob, ri, pl.ds(ci, 16)] = acc

        for b in range(NBUF):
            start_in(b, b)

        def main(cq, carry):
            for b in range(NBUF):
                c = NBUF * cq + b
                ob = b % OBUF
                wait_in(c, b)

                @pl.when(c >= OBUF)
                def _():
                    wait_out(c - OBUF, ob)

                compute(b, ob)
                start_out(c, ob)

                @pl.when(c + NBUF < CHUNKS)
                def _():
                    start_in(c + NBUF, b)
            return carry

        lax.fori_loop(0, CHUNKS // NBUF, main, 0)
        wait_out(CHUNKS - 2, (CHUNKS - 2) % OBUF)
        wait_out(CHUNKS - 1, (CHUNKS - 1) % OBUF)

    pltpu.sync_copy(t01_ref, tab_ref)
    run_phase((0, 1), False)

    pltpu.sync_copy(t23_ref, tab_ref)
    run_phase((2, 3), True)

    pltpu.sync_copy(t4_ref, tab_ref)
    run_phase((4,), True)


def _sc_gather(ep_t, t01, t23, t4):
    mesh = plsc.VectorSubcoreMesh(core_axis_name="c", subcore_axis_name="s")
    kern = functools.partial(
        pl.kernel,
        mesh=mesh,
        compiler_params=pltpu.CompilerParams(needs_layout_passes=False),
        out_type=jax.ShapeDtypeStruct((N, N), jnp.float32),
        scratch_types=[
            pltpu.VMEM((E,), jnp.int32),
            pltpu.VMEM((NBUF, 2, 8, LC), jnp.int32),
            pltpu.VMEM((NBUF, 8, LC), jnp.float32),
            pltpu.VMEM((OBUF, 8, LC), jnp.float32),
            pltpu.SemaphoreType.DMA,
            pltpu.SemaphoreType.DMA,
            pltpu.SemaphoreType.DMA,
            pltpu.SemaphoreType.DMA,
            pltpu.SemaphoreType.DMA,
            pltpu.SemaphoreType.DMA,
        ],
    )(_sc_body)
    return kern(ep_t, t01, t23, t4)


def kernel(x, edge_attr, edge_paths, edge_vector):
    assert edge_attr.shape == (E, D) and edge_paths.shape == (N, N, L)
    ev_pad = jnp.zeros((8, D), jnp.float32).at[:L].set(edge_vector / L)
    scores_t = _tc_scores(ev_pad, edge_attr)

    u = lax.bitcast_convert_type(
        scores_t.astype(jnp.bfloat16), jnp.uint16).astype(jnp.uint32)
    w01 = jnp.array([1, 1 << 16, 0, 0, 0, 0, 0, 0], jnp.uint32)
    w23 = jnp.array([0, 0, 1, 1 << 16, 0, 0, 0, 0], jnp.uint32)
    e4 = jnp.array([0, 0, 0, 0, 1, 0, 0, 0], jnp.float32)
    t01 = lax.bitcast_convert_type((u * w01[:, None]).sum(0), jnp.int32)
    t23 = lax.bitcast_convert_type((u * w23[:, None]).sum(0), jnp.int32)
    t4 = lax.bitcast_convert_type((scores_t * e4[:, None]).sum(0), jnp.int32)

    ep_t = jnp.transpose(edge_paths, (2, 0, 1))
    return _sc_gather(ep_t, t01, t23, t4)
